# baseline (speedup 1.0000x reference)
import jax
import jax.numpy as jnp
from jax.experimental import pallas as pl

_IB = 200


def _body(x_ref, c_ref, o_ref):
    o_ref[...] = jnp.sum(x_ref[...] * c_ref[...][:, :, None], axis=1)


def kernel(x, coef):
    num_trips, num_items, num_params = x.shape
    xt = jnp.transpose(x, (1, 2, 0))
    tbc = num_trips // 2
    outT = pl.pallas_call(
        _body,
        grid=(pl.cdiv(num_items, _IB), 2),
        in_specs=[
            pl.BlockSpec((_IB, num_params, tbc), lambda i, j: (i, 0, j)),
            pl.BlockSpec((_IB, num_params), lambda i, j: (i, 0)),
        ],
        out_specs=pl.BlockSpec((_IB, tbc), lambda i, j: (i, j)),
        out_shape=jax.ShapeDtypeStruct((num_items, num_trips), jnp.float32),
    )(xt, coef)
    return outT.T


# speedup vs baseline: 1.0083x; 1.0083x over previous
import jax
import jax.numpy as jnp
from jax.experimental import pallas as pl

_IB = 168


def _body(x_ref, c_ref, o_ref):
    o_ref[...] = jnp.sum(x_ref[...] * c_ref[...][:, :, None], axis=1)


def kernel(x, coef):
    num_trips, num_items, num_params = x.shape
    xt = jnp.transpose(x, (1, 2, 0))
    tbc = num_trips // 2
    outT = pl.pallas_call(
        _body,
        grid=(pl.cdiv(num_items, _IB), 2),
        in_specs=[
            pl.BlockSpec((_IB, num_params, tbc), lambda i, j: (i, 0, j)),
            pl.BlockSpec((_IB, num_params), lambda i, j: (i, 0)),
        ],
        out_specs=pl.BlockSpec((_IB, tbc), lambda i, j: (i, j)),
        out_shape=jax.ShapeDtypeStruct((num_items, num_trips), jnp.float32),
    )(xt, coef)
    return outT.T


# speedup vs baseline: 1.0174x; 1.0090x over previous
import jax
import jax.numpy as jnp
from jax.experimental import pallas as pl

_IB = 128


def _body(x_ref, c_ref, o_ref):
    o_ref[...] = jnp.sum(x_ref[...] * c_ref[...][:, :, None], axis=1)


def kernel(x, coef):
    num_trips, num_items, num_params = x.shape
    xt = jnp.transpose(x, (1, 2, 0))
    tbc = num_trips // 2
    outT = pl.pallas_call(
        _body,
        grid=(pl.cdiv(num_items, _IB), 2),
        in_specs=[
            pl.BlockSpec((_IB, num_params, tbc), lambda i, j: (i, 0, j)),
            pl.BlockSpec((_IB, num_params), lambda i, j: (i, 0)),
        ],
        out_specs=pl.BlockSpec((_IB, tbc), lambda i, j: (i, j)),
        out_shape=jax.ShapeDtypeStruct((num_items, num_trips), jnp.float32),
    )(xt, coef)
    return outT.T


# speedup vs baseline: 1.0175x; 1.0002x over previous
import jax
import jax.numpy as jnp
from jax.experimental import pallas as pl
from jax.experimental.pallas import tpu as pltpu

_IB = 128


def _body(x_ref, c_ref, o_ref):
    o_ref[...] = jnp.sum(x_ref[...] * c_ref[...][:, :, None], axis=1)


def kernel(x, coef):
    num_trips, num_items, num_params = x.shape
    xt = jnp.transpose(x, (1, 2, 0))
    tbc = num_trips // 2
    outT = pl.pallas_call(
        _body,
        grid=(pl.cdiv(num_items, _IB), 2),
        in_specs=[
            pl.BlockSpec((_IB, num_params, tbc), lambda i, j: (i, 0, j)),
            pl.BlockSpec((_IB, num_params), lambda i, j: (i, 0)),
        ],
        out_specs=pl.BlockSpec((_IB, tbc), lambda i, j: (i, j)),
        out_shape=jax.ShapeDtypeStruct((num_items, num_trips), jnp.float32),
        compiler_params=pltpu.CompilerParams(
            dimension_semantics=("parallel", "arbitrary")),
    )(xt, coef)
    return outT.T
